# baseline (speedup 1.0000x reference)
import functools

import jax
import jax.numpy as jnp
from jax import lax
from jax.experimental import pallas as pl
from jax.experimental.pallas import tpu as pltpu
from jax.experimental.pallas import tpu_sc as plsc

ROWS = 1024
COLS = 100000
K = 200

NC, NS, L = 2, 16, 16
NW = NC * NS
CHUNK = 128
NCG = 8
NKG = 4
K_START = (0, 56, 104, 152)
K_MAX = 56

_mesh = plsc.VectorSubcoreMesh(core_axis_name="c", subcore_axis_name="s")


@functools.partial(
    pl.kernel,
    mesh=_mesh,
    out_type=jax.ShapeDtypeStruct((K, ROWS), jnp.float32),
    compiler_params=pltpu.CompilerParams(disable_bounds_checks=True),
    scratch_types=[
        pltpu.VMEM((K_MAX, CHUNK), jnp.int32),
        pltpu.VMEM((K_MAX, CHUNK), jnp.int32),
        pltpu.VMEM((K_MAX, CHUNK), jnp.float32),
        pltpu.SemaphoreType.DMA,
        pltpu.SemaphoreType.DMA,
    ],
)
def _sc_gather(xt_hbm, idx_hbm, out_hbm, idx_v, gidx_v, vals_v, sem, isem):
    wid = lax.axis_index("s") * NC + lax.axis_index("c")
    cg = wid & (NCG - 1)
    kg = wid >> 3
    k0 = (kg > 0) * 8 + kg * 48
    klen = jnp.where(kg == 0, K_MAX, 48)
    nblk = klen >> 3

    @pl.loop(0, nblk)
    def _(g):
        pltpu.async_copy(
            idx_hbm.at[pl.ds(k0 + g * 8, 8), pl.ds(cg * CHUNK, CHUNK)],
            idx_v.at[pl.ds(g * 8, 8)],
            isem,
        )

    lanes = lax.broadcasted_iota(jnp.int32, (L,), 0)

    x_flat = xt_hbm.at[0, pl.ds(0, CHUNK)]

    @pl.loop(0, nblk)
    def _(g):
        pltpu.make_async_copy(
            idx_hbm.at[pl.ds(k0 + g * 8, 8), pl.ds(cg * CHUNK, CHUNK)],
            idx_v.at[pl.ds(g * 8, 8)],
            isem,
        ).wait()

        @pl.loop(g * 8, g * 8 + 8)
        def _(r):
            for c in range(CHUNK // L):
                icontrib = cg * 1024 + c * L + lanes
                j = idx_v[r, pl.ds(c * L, L)]
                gidx_v[r, pl.ds(c * L, L)] = (
                    ((j >> 3) << 13) + ((j & 7) << 7) + icontrib
                )
            pltpu.async_copy(
                x_flat.at[gidx_v.at[r, pl.ds(0, 64)]],
                vals_v.at[r, pl.ds(0, 64)],
                sem,
            )
            pltpu.async_copy(
                x_flat.at[gidx_v.at[r, pl.ds(64, 64)]],
                vals_v.at[r, pl.ds(64, 64)],
                sem,
            )

    pltpu.make_async_copy(
        xt_hbm.at[pl.ds(0, 48), pl.ds(0, CHUNK)],
        vals_v.at[pl.ds(0, 48)],
        sem,
    ).wait()

    @pl.when(kg == 0)
    def _():
        pltpu.make_async_copy(
            xt_hbm.at[pl.ds(48, 8), pl.ds(0, CHUNK)],
            vals_v.at[pl.ds(48, 8)],
            sem,
        ).wait()

    pltpu.sync_copy(
        vals_v.at[pl.ds(0, 48)],
        out_hbm.at[pl.ds(k0, 48), pl.ds(cg * CHUNK, CHUNK)],
    )

    @pl.when(kg == 0)
    def _():
        pltpu.sync_copy(
            vals_v.at[pl.ds(48, 8)],
            out_hbm.at[pl.ds(48, 8), pl.ds(cg * CHUNK, CHUNK)],
        )


def kernel(x, indices):
    out = _sc_gather(x.T, indices.T)
    return out.T


# speedup vs baseline: 1.0036x; 1.0036x over previous
import functools

import jax
import jax.numpy as jnp
from jax import lax
from jax.experimental import pallas as pl
from jax.experimental.pallas import tpu as pltpu
from jax.experimental.pallas import tpu_sc as plsc

ROWS = 1024
COLS = 100000
K = 200

NC, NS, L = 2, 16, 16
NW = NC * NS
CHUNK = 128
NCG = 8
NKG = 4
K_START = (0, 56, 104, 152)
K_MAX = 56

_mesh = plsc.VectorSubcoreMesh(core_axis_name="c", subcore_axis_name="s")


@functools.partial(
    pl.kernel,
    mesh=_mesh,
    out_type=jax.ShapeDtypeStruct((K, ROWS), jnp.float32),
    compiler_params=pltpu.CompilerParams(disable_bounds_checks=True),
    scratch_types=[
        pltpu.VMEM((K_MAX, CHUNK), jnp.int32),
        pltpu.VMEM((K_MAX, CHUNK), jnp.int32),
        pltpu.VMEM((K_MAX, CHUNK), jnp.float32),
        pltpu.SemaphoreType.DMA,
        pltpu.SemaphoreType.DMA,
    ],
)
def _sc_gather(xt_hbm, idx_hbm, out_hbm, idx_v, gidx_v, vals_v, sem, isem):
    wid = lax.axis_index("s") * NC + lax.axis_index("c")
    cg = wid & (NCG - 1)
    kg = wid >> 3
    k0 = (kg > 0) * 8 + kg * 48
    klen = jnp.where(kg == 0, K_MAX, 48)
    nblk = klen >> 3

    @pl.loop(0, nblk)
    def _(g):
        pltpu.async_copy(
            idx_hbm.at[pl.ds(k0 + g * 8, 8), pl.ds(cg * CHUNK, CHUNK)],
            idx_v.at[pl.ds(g * 8, 8)],
            isem,
        )

    lanes = lax.broadcasted_iota(jnp.int32, (L,), 0)

    x_flat = xt_hbm.at[0, pl.ds(0, CHUNK)]

    @pl.loop(0, nblk)
    def _(g):
        pltpu.make_async_copy(
            idx_hbm.at[pl.ds(k0 + g * 8, 8), pl.ds(cg * CHUNK, CHUNK)],
            idx_v.at[pl.ds(g * 8, 8)],
            isem,
        ).wait()

        @pl.loop(g * 8, g * 8 + 8)
        def _(r):
            for c in range(CHUNK // L):
                icontrib = cg * 1024 + c * L + lanes
                j = idx_v[r, pl.ds(c * L, L)]
                gidx_v[r, pl.ds(c * L, L)] = (
                    ((j >> 3) << 13) + ((j & 7) << 7) + icontrib
                )
            pltpu.async_copy(x_flat.at[gidx_v.at[r]], vals_v.at[r], sem)

    pltpu.make_async_copy(
        xt_hbm.at[pl.ds(0, 48), pl.ds(0, CHUNK)],
        vals_v.at[pl.ds(0, 48)],
        sem,
    ).wait()

    @pl.when(kg == 0)
    def _():
        pltpu.make_async_copy(
            xt_hbm.at[pl.ds(48, 8), pl.ds(0, CHUNK)],
            vals_v.at[pl.ds(48, 8)],
            sem,
        ).wait()

    pltpu.sync_copy(
        vals_v.at[pl.ds(0, 48)],
        out_hbm.at[pl.ds(k0, 48), pl.ds(cg * CHUNK, CHUNK)],
    )

    @pl.when(kg == 0)
    def _():
        pltpu.sync_copy(
            vals_v.at[pl.ds(48, 8)],
            out_hbm.at[pl.ds(48, 8), pl.ds(cg * CHUNK, CHUNK)],
        )


def kernel(x, indices):
    out = _sc_gather(x.T, indices.T)
    return out.T
